# baseline (speedup 1.0000x reference)
import jax
import jax.numpy as jnp
from jax import lax
from jax.experimental import pallas as pl
from jax.experimental.pallas import tpu as pltpu
from jax.experimental.pallas import tpu_sc as plsc

B = 1024
N_SRC = 4096
HW = 64 * 64
ROW = 2 * HW
NC = 2
NS = 16
NW = NC * NS
SYS_PER = B // NW
SEG_WIN = 48
L = 16
CG = 8
GROUPS = HW // (CG * L)


def _scalar_at(ref, j):
    v = plsc.load_gather(ref, [jnp.broadcast_to(j, (L,)).astype(jnp.int32)])
    return jnp.max(v)


def _sc_body(grid_hbm, params_hbm, order_hbm, seg_hbm, out_hbm,
             seg_v, order_v, params_v, grid_v, acc_v):
    wid = lax.axis_index("s") * NC + lax.axis_index("c")
    pltpu.sync_copy(order_hbm, order_v)
    pltpu.sync_copy(params_hbm, params_v)
    pltpu.sync_copy(seg_hbm.at[pl.ds(wid * SYS_PER, SEG_WIN)], seg_v)

    iota = lax.iota(jnp.int32, L)
    two_iota = iota * 2
    zeros = jnp.zeros((L,), jnp.float32)

    def do_system(i, carry):
        b = wid * SYS_PER + i
        pltpu.sync_copy(grid_hbm.at[b], grid_v)

        svec = plsc.load_gather(seg_v, [(i + iota).astype(jnp.int32)])
        s0 = jnp.max(jnp.where(iota == 0, svec, 0))
        s1 = jnp.max(jnp.where(iota == 1, svec, 0))

        for g in range(GROUPS):
            gxs, gys = [], []
            for c in range(CG):
                ix = two_iota + (g * CG + c) * 2 * L
                gxs.append(plsc.load_gather(grid_v, [ix]))
                gys.append(plsc.load_gather(grid_v, [ix + 1]))

            def do_source(s, accs, gxs=gxs, gys=gys):
                sidv = plsc.load_gather(
                    order_v, [jnp.broadcast_to(s, (L,)).astype(jnp.int32)]
                )
                p = sidv * 4
                x0 = plsc.load_gather(params_v, [p])
                y0 = plsc.load_gather(params_v, [p + 1])
                amp = plsc.load_gather(params_v, [p + 2])
                sg = plsc.load_gather(params_v, [p + 3])
                nk = -0.5 / (sg * sg)
                out = []
                for c in range(CG):
                    dx = gxs[c] - x0
                    dy = gys[c] - y0
                    out.append(
                        accs[c] + amp * jnp.exp((dx * dx + dy * dy) * nk)
                    )
                return tuple(out)

            accs = lax.fori_loop(s0, s1, do_source, (zeros,) * CG)
            for c in range(CG):
                acc_v[pl.ds((g * CG + c) * L, L)] = accs[c]

        pltpu.sync_copy(acc_v, out_hbm.at[b])
        return carry
    lax.fori_loop(0, SYS_PER, do_system, 0)


def kernel(source_grid, blob_params, sys_idx):
    source_grid = source_grid.astype(jnp.float32)
    idx = sys_idx.astype(jnp.int32)
    order = jnp.argsort(idx).astype(jnp.int32)
    sorted_sys = jnp.sort(idx)
    seg = jnp.searchsorted(
        sorted_sys, jnp.arange(B + 1, dtype=jnp.int32), side="left"
    ).astype(jnp.int32)
    seg = jnp.concatenate(
        [seg, jnp.full((NW * SYS_PER + SEG_WIN - (B + 1),), N_SRC, jnp.int32)]
    )

    grid2 = source_grid.reshape(B, ROW)
    params_flat = blob_params.astype(jnp.float32).reshape(-1)

    mesh = plsc.VectorSubcoreMesh(core_axis_name="c", subcore_axis_name="s")
    run = pl.kernel(
        _sc_body,
        mesh=mesh,
        compiler_params=pltpu.CompilerParams(needs_layout_passes=False),
        out_type=jax.ShapeDtypeStruct((B, HW), jnp.float32),
        scratch_types=[
            pltpu.VMEM((SEG_WIN,), jnp.int32),
            pltpu.VMEM((N_SRC,), jnp.int32),
            pltpu.VMEM((4 * N_SRC,), jnp.float32),
            pltpu.VMEM((ROW,), jnp.float32),
            pltpu.VMEM((HW,), jnp.float32),
        ],
    )
    out = run(grid2, params_flat, order, seg)
    return out.reshape(B, 64, 64)


# speedup vs baseline: 1.5938x; 1.5938x over previous
import jax
import jax.numpy as jnp
from jax import lax
from jax.experimental import pallas as pl
from jax.experimental.pallas import tpu as pltpu
from jax.experimental.pallas import tpu_sc as plsc

B = 1024
N_SRC = 4096
HW = 64 * 64
ROW = 2 * HW
NC = 2
NS = 16
NW = NC * NS
SYS_PER = B // NW
SEG_WIN = 48
L = 16
CG = 8
GROUPS = HW // (CG * L)


def _scalar_at(ref, j):
    v = plsc.load_gather(ref, [jnp.broadcast_to(j, (L,)).astype(jnp.int32)])
    return jnp.max(v)


def _sc_body(grid_hbm, params_hbm, order_hbm, seg_hbm, out_hbm,
             seg_v, order_v, params_v, grid_v, acc_v, gx_v, gy_v):
    wid = lax.axis_index("s") * NC + lax.axis_index("c")
    pltpu.sync_copy(order_hbm, order_v)
    pltpu.sync_copy(params_hbm, params_v)
    pltpu.sync_copy(seg_hbm.at[pl.ds(wid * SYS_PER, SEG_WIN)], seg_v)

    iota = lax.iota(jnp.int32, L)
    two_iota = iota * 2
    zeros = jnp.zeros((L,), jnp.float32)

    def do_system(i, carry):
        b = wid * SYS_PER + i
        pltpu.sync_copy(grid_hbm.at[b], grid_v)

        svec = plsc.load_gather(seg_v, [(i + iota).astype(jnp.int32)])
        s0 = jnp.max(jnp.where(iota == 0, svec, 0))
        s1 = jnp.max(jnp.where(iota == 1, svec, 0))

        @plsc.parallel_loop(0, HW // L, unroll=8)
        def zero_chunk(k):
            acc_v[pl.ds(k * L, L)] = zeros

        @pl.when(s1 > s0)
        def _nonempty():
            @plsc.parallel_loop(0, HW // L, unroll=8)
            def deint(k):
                ix = two_iota + k * 2 * L
                gx_v[pl.ds(k * L, L)] = plsc.load_gather(grid_v, [ix])
                gy_v[pl.ds(k * L, L)] = plsc.load_gather(grid_v, [ix + 1])

            def do_source(s, c):
                sidv = plsc.load_gather(
                    order_v, [jnp.broadcast_to(s, (L,)).astype(jnp.int32)]
                )
                p = sidv * 4
                x0 = plsc.load_gather(params_v, [p])
                y0 = plsc.load_gather(params_v, [p + 1])
                amp = plsc.load_gather(params_v, [p + 2])
                sg = plsc.load_gather(params_v, [p + 3])
                nk = -0.5 / (sg * sg)

                @plsc.parallel_loop(0, HW // L, unroll=8)
                def do_chunk(k):
                    gx = gx_v[pl.ds(k * L, L)]
                    gy = gy_v[pl.ds(k * L, L)]
                    dx = gx - x0
                    dy = gy - y0
                    val = amp * jnp.exp((dx * dx + dy * dy) * nk)
                    plsc.addupdate(acc_v.at[pl.ds(k * L, L)], val)
                return c
            lax.fori_loop(s0, s1, do_source, 0)

        pltpu.sync_copy(acc_v, out_hbm.at[b])
        return carry
    lax.fori_loop(0, SYS_PER, do_system, 0)


def kernel(source_grid, blob_params, sys_idx):
    source_grid = source_grid.astype(jnp.float32)
    idx = sys_idx.astype(jnp.int32)
    order = jnp.argsort(idx).astype(jnp.int32)
    sorted_sys = jnp.sort(idx)
    seg = jnp.searchsorted(
        sorted_sys, jnp.arange(B + 1, dtype=jnp.int32), side="left"
    ).astype(jnp.int32)
    seg = jnp.concatenate(
        [seg, jnp.full((NW * SYS_PER + SEG_WIN - (B + 1),), N_SRC, jnp.int32)]
    )

    grid2 = source_grid.reshape(B, ROW)
    params_flat = blob_params.astype(jnp.float32).reshape(-1)

    mesh = plsc.VectorSubcoreMesh(core_axis_name="c", subcore_axis_name="s")
    run = pl.kernel(
        _sc_body,
        mesh=mesh,
        compiler_params=pltpu.CompilerParams(needs_layout_passes=False),
        out_type=jax.ShapeDtypeStruct((B, HW), jnp.float32),
        scratch_types=[
            pltpu.VMEM((SEG_WIN,), jnp.int32),
            pltpu.VMEM((N_SRC,), jnp.int32),
            pltpu.VMEM((4 * N_SRC,), jnp.float32),
            pltpu.VMEM((ROW,), jnp.float32),
            pltpu.VMEM((HW,), jnp.float32),
            pltpu.VMEM((HW,), jnp.float32),
            pltpu.VMEM((HW,), jnp.float32),
        ],
    )
    out = run(grid2, params_flat, order, seg)
    return out.reshape(B, 64, 64)
